# Initial kernel scaffold; baseline (speedup 1.0000x reference)
import jax
import jax.numpy as jnp
from jax.experimental import pallas as pl


def kernel(hidden_states, topk_weights, topk_ids, gate_up_weight, down_weight):
    raise NotImplementedError("write your pallas kernel here")


# speedup vs baseline: 1.2614x; 1.2614x over previous
import functools

import jax
import jax.numpy as jnp
from jax import lax
from jax.experimental import pallas as pl
from jax.experimental.pallas import tpu as pltpu
from jax.experimental.pallas import tpu_sc as plsc

H = 768
F = 512
E = 64
K = 2
C = 192
T = 2048
A = T * K

RB = 64
BPE = C // RB
NB = E * BPE + 1
ROWS = NB * RB
DUMP = E * C

NC = 2
NS = 16
NW = NC * NS
APW = A // NW
TPW = T // NW
LANES = 16


def _dispatch_body(hs_hbm, tok_hbm, dest_hbm, out_hbm, tok_v, dest_v, rows_v, sem):
    wid = lax.axis_index("s") * NC + lax.axis_index("c")
    base = wid * APW
    pltpu.sync_copy(tok_hbm.at[pl.ds(base, APW)], tok_v)
    pltpu.sync_copy(dest_hbm.at[pl.ds(base, APW)], dest_v)
    pltpu.async_copy(hs_hbm.at[tok_v], rows_v, sem).wait()
    pltpu.async_copy(rows_v, out_hbm.at[dest_v], sem).wait()


def _combine_body(eo_hbm, ia_hbm, ib_hbm, w_hbm, out_hbm,
                  ia_v, ib_v, w_v, buf_a, buf_b, sem_a, sem_b):
    wid = lax.axis_index("s") * NC + lax.axis_index("c")
    tb = wid * TPW
    pltpu.sync_copy(ia_hbm.at[pl.ds(tb, TPW)], ia_v)
    pltpu.sync_copy(ib_hbm.at[pl.ds(tb, TPW)], ib_v)
    pltpu.sync_copy(w_hbm.at[pl.ds(K * tb, K * TPW)], w_v)
    ca = pltpu.async_copy(eo_hbm.at[ia_v], buf_a, sem_a)
    cb = pltpu.async_copy(eo_hbm.at[ib_v], buf_b, sem_b)
    ca.wait()
    cb.wait()

    def group_body(g, _):
        wpair = w_v[pl.ds(g * LANES, LANES)]
        for j in range(LANES // K):
            i = g * (LANES // K) + j
            w0 = wpair[K * j]
            w1 = wpair[K * j + 1]

            def chunk_body(c, _, i=i, w0=w0, w1=w1):
                a = buf_a[i, pl.ds(c * LANES, LANES)]
                b = buf_b[i, pl.ds(c * LANES, LANES)]
                buf_a[i, pl.ds(c * LANES, LANES)] = w0 * a + w1 * b
                return 0

            lax.fori_loop(0, H // LANES, chunk_body, 0)
        return 0

    lax.fori_loop(0, TPW // (LANES // K), group_body, 0)
    pltpu.sync_copy(buf_a, out_hbm.at[pl.ds(tb, TPW)])


_SC_MESH = plsc.VectorSubcoreMesh(core_axis_name="c", subcore_axis_name="s")

_dispatch = functools.partial(
    pl.kernel,
    mesh=_SC_MESH,
    out_type=jax.ShapeDtypeStruct((ROWS, H), jnp.float32),
    scratch_types=[
        pltpu.VMEM((APW,), jnp.int32),
        pltpu.VMEM((APW,), jnp.int32),
        pltpu.VMEM((APW, H), jnp.float32),
        pltpu.SemaphoreType.DMA,
    ],
)(_dispatch_body)

_combine = functools.partial(
    pl.kernel,
    mesh=_SC_MESH,
    out_type=jax.ShapeDtypeStruct((T, H), jnp.float32),
    scratch_types=[
        pltpu.VMEM((TPW,), jnp.int32),
        pltpu.VMEM((TPW,), jnp.int32),
        pltpu.VMEM((K * TPW,), jnp.float32),
        pltpu.VMEM((TPW, H), jnp.float32),
        pltpu.VMEM((TPW, H), jnp.float32),
        pltpu.SemaphoreType.DMA,
        pltpu.SemaphoreType.DMA,
    ],
)(_combine_body)


def _gemm_body(counts_ref, x_ref, gu_ref, dw_ref, o_ref):
    b = pl.program_id(0)

    @pl.when(b == NB - 1)
    def _zero():
        o_ref[...] = jnp.zeros_like(o_ref)

    @pl.when(b < NB - 1)
    def _compute():
        e = b // BPE
        rb = b % BPE

        @pl.when(rb * RB < counts_ref[e])
        def _active():
            x = x_ref[...]
            gu = gu_ref[0]
            acc = lax.dot_general(x, gu, (((1,), (1,)), ((), ())),
                                  preferred_element_type=jnp.float32)
            gate = acc[:, :F]
            up = acc[:, F:]
            act = gate * jax.nn.sigmoid(gate) * up
            dw = dw_ref[0]
            o_ref[...] = lax.dot_general(act, dw, (((1,), (1,)), ((), ())),
                                         preferred_element_type=jnp.float32)


def _weight_index(b, counts):
    e = jnp.minimum(b // BPE, E - 1)
    return (e, 0, 0)


_gemm = pl.pallas_call(
    _gemm_body,
    grid_spec=pltpu.PrefetchScalarGridSpec(
        num_scalar_prefetch=1,
        grid=(NB,),
        in_specs=[
            pl.BlockSpec((RB, H), lambda b, counts: (b, 0)),
            pl.BlockSpec((1, 2 * F, H), _weight_index),
            pl.BlockSpec((1, H, F), _weight_index),
        ],
        out_specs=pl.BlockSpec((RB, H), lambda b, counts: (b, 0)),
    ),
    out_shape=jax.ShapeDtypeStruct((ROWS, H), jnp.float32),
    compiler_params=pltpu.CompilerParams(
        dimension_semantics=("arbitrary",),
    ),
)


@jax.jit
def kernel(hidden_states, topk_weights, topk_ids, gate_up_weight, down_weight):
    flat_ids = topk_ids.reshape(-1).astype(jnp.int32)
    flat_w = topk_weights.reshape(-1)

    onehot = (flat_ids[:, None] == jnp.arange(E, dtype=jnp.int32)[None, :])
    onehot = onehot.astype(jnp.int32)
    pos = jnp.cumsum(onehot, axis=0) - 1
    pos_in_expert = jnp.take_along_axis(pos, flat_ids[:, None], axis=1)[:, 0]
    valid = pos_in_expert < C
    dest = jnp.where(valid, flat_ids * C + pos_in_expert, DUMP).astype(jnp.int32)
    counts = jnp.minimum(jnp.sum(onehot, axis=0), C).astype(jnp.int32)
    token_idx = (jnp.arange(A, dtype=jnp.int32) // K).astype(jnp.int32)

    expert_in = _dispatch(hidden_states, token_idx, dest)
    eo = _gemm(counts, expert_in, gate_up_weight, down_weight)
    idx_a = dest.reshape(T, K)[:, 0]
    idx_b = dest.reshape(T, K)[:, 1]
    return _combine(eo, idx_a, idx_b, flat_w)
